# baseline (speedup 1.0000x reference)
import functools

import jax
import jax.numpy as jnp
from jax import lax
from jax.experimental import pallas as pl
from jax.experimental.pallas import tpu as pltpu
from jax.experimental.pallas import tpu_sc as plsc

B, C, N_IN, N_OUT, K = 2, 256, 50000, 12500, 8
R = B * C
NP = 12512
G = NP // 32
NW = 32
ROWS_PER_W = R // NW
STAGE = 25024


def _body(x_hbm, pidx_hbm, out_hbm, row_v, idx_v, stage_v):
    wid = lax.axis_index("c") * 16 + lax.axis_index("s")

    pltpu.sync_copy(pidx_hbm, idx_v)
    lanes = lax.iota(jnp.int32, 16)

    for r in range(ROWS_PER_W):
        row_id = wid * ROWS_PER_W + r
        pltpu.sync_copy(x_hbm.at[row_id], row_v)
        half = (r % 2) * N_OUT

        @plsc.parallel_loop(0, G, unroll=2)
        def do_group(g):
            acc_a = None
            acc_b = None
            for k in range(K):
                p = idx_v[k, pl.ds(g * 16, 16)]
                a = p & 0xFFFF
                b = (p >> 16) & 0xFFFF
                va = plsc.load_gather(row_v, [a])
                vb = plsc.load_gather(row_v, [b])
                acc_a = va if acc_a is None else jnp.maximum(acc_a, va)
                acc_b = vb if acc_b is None else jnp.maximum(acc_b, vb)
            base = g * 32 + half + lanes
            plsc.store_scatter(stage_v, [base], acc_a)
            plsc.store_scatter(stage_v, [base + 16], acc_b)

        if r % 2 == 1:
            flat = (wid * ROWS_PER_W + r - 1) * N_OUT
            pltpu.sync_copy(
                stage_v.at[pl.ds(0, 2 * N_OUT)],
                out_hbm.at[pl.ds(flat, 2 * N_OUT)],
            )


_sc_call = functools.partial(
    pl.kernel,
    out_type=jax.ShapeDtypeStruct((R * N_OUT,), jnp.float32),
    mesh=plsc.VectorSubcoreMesh(core_axis_name="c", subcore_axis_name="s"),
    compiler_params=pltpu.CompilerParams(needs_layout_passes=False),
    scratch_types=[
        pltpu.VMEM((N_IN,), jnp.float32),
        pltpu.VMEM((K, NP // 2), jnp.int32),
        pltpu.VMEM((STAGE,), jnp.float32),
    ],
)(_body)


def kernel(x, idx):
    xr = x.reshape(R, N_IN)
    idxp = jnp.concatenate([idx, jnp.zeros((NP - N_OUT, K), jnp.int32)], axis=0)
    t = idxp.T.reshape(K, G, 2, 16)
    packed = (t[:, :, 0, :] | (t[:, :, 1, :] << 16)).reshape(K, NP // 2)
    out = _sc_call(xr, packed)
    return out.reshape(B, C, N_OUT)


# speedup vs baseline: 1.5449x; 1.5449x over previous
import functools

import jax
import jax.numpy as jnp
from jax import lax
from jax.experimental import pallas as pl
from jax.experimental.pallas import tpu as pltpu
from jax.experimental.pallas import tpu_sc as plsc

B, C, N_IN, N_OUT, K = 2, 256, 50000, 12500, 8
NW = 32
NPAD = 12544
PPT = NPAD // NW
CP = 4
NCH = PPT // CP
ROWS = CP * B * K
SLAB = B * C


def _body(tab_hbm, idxg_hbm, out_hbm, idx_v, rows_v, stage_v, gsem, osem):
    wid = lax.axis_index("c") * 16 + lax.axis_index("s")
    start_pt = wid * PPT

    pltpu.sync_copy(idxg_hbm.at[pl.ds(start_pt * B * K, PPT * B * K)], idx_v)

    def gather(c, buf):
        pltpu.async_copy(
            tab_hbm.at[idx_v.at[pl.ds(c * ROWS, ROWS)]],
            rows_v.at[buf],
            gsem.at[buf],
        )

    def gwait(c, buf):
        pltpu.make_async_copy(
            tab_hbm.at[idx_v.at[pl.ds(c * ROWS, ROWS)]],
            rows_v.at[buf],
            gsem.at[buf],
        ).wait()

    def compute(c, buf):
        def do_v(v, _):
            coff = (v // 8) * 256 + (v % 8) * 16
            for p in range(CP):
                for b in range(B):
                    acc = None
                    for k in range(K):
                        r = rows_v[buf, p * (B * K) + b * K + k, pl.ds(v * 16, 16)]
                        acc = r if acc is None else jnp.maximum(acc, r)
                    stage_v[buf, pl.ds(p * SLAB + b * 128 + coff, 16)] = acc
            return _

        lax.fori_loop(0, 16, do_v, 0, unroll=2)
        pltpu.async_copy(
            stage_v.at[buf],
            out_hbm.at[pl.ds((start_pt + c * CP) * SLAB, CP * SLAB)],
            osem.at[buf],
        )

    def owait(c, buf):
        pltpu.make_async_copy(
            stage_v.at[buf],
            out_hbm.at[pl.ds((start_pt + c * CP) * SLAB, CP * SLAB)],
            osem.at[buf],
        ).wait()

    ncr = jnp.minimum(jnp.maximum((N_OUT - start_pt) // CP, 0), NCH)

    @pl.when(ncr > 0)
    def _prologue():
        gather(0, 0)

    def step(c, _):
        buf = lax.rem(c, 2)
        nbuf = 1 - buf

        @pl.when(c + 1 < ncr)
        def _prefetch():
            gather(c + 1, nbuf)

        gwait(c, buf)

        @pl.when(c >= 2)
        def _drain_out():
            owait(c - 2, buf)

        compute(c, buf)
        return _

    lax.fori_loop(0, ncr, step, 0)

    @pl.when(ncr >= 2)
    def _drain_m2():
        owait(ncr - 2, lax.rem(ncr - 2, 2))

    @pl.when(ncr >= 1)
    def _drain_m1():
        owait(ncr - 1, lax.rem(ncr - 1, 2))


_sc_call = functools.partial(
    pl.kernel,
    out_type=jax.ShapeDtypeStruct((N_OUT * SLAB,), jnp.float32),
    mesh=plsc.VectorSubcoreMesh(core_axis_name="c", subcore_axis_name="s"),
    compiler_params=pltpu.CompilerParams(
        needs_layout_passes=False, use_tc_tiling_on_sc=True
    ),
    scratch_types=[
        pltpu.VMEM((PPT * B * K,), jnp.int32),
        pltpu.VMEM((2, ROWS, C), jnp.float32),
        pltpu.VMEM((2, CP * SLAB), jnp.float32),
        pltpu.SemaphoreType.DMA((2,)),
        pltpu.SemaphoreType.DMA((2,)),
    ],
)(_body)


def kernel(x, idx):
    tab = x.transpose(0, 2, 1).reshape(B * N_IN, C)
    idxp = jnp.concatenate(
        [idx, jnp.zeros((NPAD - N_OUT, K), jnp.int32)], axis=0
    )
    boff = jnp.arange(B, dtype=jnp.int32)[None, :, None] * N_IN
    idxg = (idxp[:, None, :] + boff).reshape(-1)
    out = _sc_call(tab, idxg)
    o4 = out.reshape(N_OUT, 2, B, 128)
    return o4.transpose(2, 1, 3, 0).reshape(B, C, N_OUT)


# speedup vs baseline: 2.0916x; 1.3539x over previous
import functools

import jax
import jax.numpy as jnp
from jax import lax
from jax.experimental import pallas as pl
from jax.experimental.pallas import tpu as pltpu
from jax.experimental.pallas import tpu_sc as plsc

B, C, N_IN, N_OUT, K = 2, 256, 50000, 12500, 8
NW = 32
NPAD = 12544
PPT = NPAD // NW
CP = 4
NCH = PPT // CP
ROWS = CP * K
SLAB = B * C


def _body(tab_hbm, idxg_hbm, out_hbm, idx_v, idx2_v, rows_v, stage_v, gsem, osem):
    wid = lax.axis_index("c") * 16 + lax.axis_index("s")
    start_pt = wid * PPT

    pltpu.sync_copy(idxg_hbm.at[pl.ds(start_pt * K, PPT * K)], idx_v)

    def shift(i, _):
        idx2_v[pl.ds(i * 16, 16)] = idx_v[pl.ds(i * 16, 16)] + N_IN
        return _

    lax.fori_loop(0, PPT * K // 16, shift, 0, unroll=4)
    idxs = (idx_v, idx2_v)

    def gather(c, buf):
        for b in range(B):
            pltpu.async_copy(
                tab_hbm.at[idxs[b].at[pl.ds(c * ROWS, ROWS)]],
                rows_v.at[buf, pl.ds(b * ROWS, ROWS)],
                gsem.at[buf],
            )

    def gwait(c, buf):
        for b in range(B):
            pltpu.make_async_copy(
                tab_hbm.at[idxs[b].at[pl.ds(c * ROWS, ROWS)]],
                rows_v.at[buf, pl.ds(b * ROWS, ROWS)],
                gsem.at[buf],
            ).wait()

    def compute(c, buf):
        def do_v(v, _):
            coff = (v // 8) * 256 + (v % 8) * 16
            for p in range(CP):
                for b in range(B):
                    acc = None
                    for k in range(K):
                        r = rows_v[buf, b * ROWS + p * K + k, pl.ds(v * 16, 16)]
                        acc = r if acc is None else jnp.maximum(acc, r)
                    stage_v[buf, pl.ds(p * SLAB + b * 128 + coff, 16)] = acc
            return _

        lax.fori_loop(0, 16, do_v, 0, unroll=2)
        pltpu.async_copy(
            stage_v.at[buf],
            out_hbm.at[pl.ds((start_pt + c * CP) * SLAB, CP * SLAB)],
            osem.at[buf],
        )

    def owait(c, buf):
        pltpu.make_async_copy(
            stage_v.at[buf],
            out_hbm.at[pl.ds((start_pt + c * CP) * SLAB, CP * SLAB)],
            osem.at[buf],
        ).wait()

    ncr = jnp.minimum(jnp.maximum((N_OUT - start_pt) // CP, 0), NCH)

    @pl.when(ncr > 0)
    def _prologue():
        gather(0, 0)

    def step(c, _):
        buf = lax.rem(c, 2)
        nbuf = 1 - buf

        @pl.when(c + 1 < ncr)
        def _prefetch():
            gather(c + 1, nbuf)

        gwait(c, buf)

        @pl.when(c >= 2)
        def _drain_out():
            owait(c - 2, buf)

        compute(c, buf)
        return _

    lax.fori_loop(0, ncr, step, 0)

    @pl.when(ncr >= 2)
    def _drain_m2():
        owait(ncr - 2, lax.rem(ncr - 2, 2))

    @pl.when(ncr >= 1)
    def _drain_m1():
        owait(ncr - 1, lax.rem(ncr - 1, 2))


_sc_call = functools.partial(
    pl.kernel,
    out_type=jax.ShapeDtypeStruct((N_OUT * SLAB,), jnp.float32),
    mesh=plsc.VectorSubcoreMesh(core_axis_name="c", subcore_axis_name="s"),
    compiler_params=pltpu.CompilerParams(
        needs_layout_passes=False, use_tc_tiling_on_sc=True
    ),
    scratch_types=[
        pltpu.VMEM((PPT * K,), jnp.int32),
        pltpu.VMEM((PPT * K,), jnp.int32),
        pltpu.VMEM((2, B * ROWS, C), jnp.float32),
        pltpu.VMEM((2, CP * SLAB), jnp.float32),
        pltpu.SemaphoreType.DMA((2,)),
        pltpu.SemaphoreType.DMA((2,)),
    ],
)(_body)


def kernel(x, idx):
    tab = x.transpose(0, 2, 1).reshape(B * N_IN, C)
    idxg = jnp.concatenate(
        [idx.reshape(-1), jnp.zeros((NPAD - N_OUT) * K, jnp.int32)]
    )
    out = _sc_call(tab, idxg)
    o4 = out.reshape(N_OUT, 2, B, 128)
    return o4.transpose(2, 1, 3, 0).reshape(B, C, N_OUT)


# speedup vs baseline: 2.1589x; 1.0322x over previous
import functools

import jax
import jax.numpy as jnp
from jax import lax
from jax.experimental import pallas as pl
from jax.experimental.pallas import tpu as pltpu
from jax.experimental.pallas import tpu_sc as plsc

B, C, N_IN, N_OUT, K = 2, 256, 50000, 12500, 8
NW = 32
NPAD = 12800
PPT = NPAD // NW
CP = 10
NCH = PPT // CP
ROWS = CP * K
SLAB = B * C


def _body(tab_hbm, idxg_hbm, out_hbm, idx_v, idx2_v, rows_v, stage_v, gsem, osem):
    wid = lax.axis_index("c") * 16 + lax.axis_index("s")
    start_pt = wid * PPT

    pltpu.sync_copy(idxg_hbm.at[pl.ds(start_pt * K, PPT * K)], idx_v)

    def shift(i, _):
        idx2_v[pl.ds(i * 16, 16)] = idx_v[pl.ds(i * 16, 16)] + N_IN
        return _

    lax.fori_loop(0, PPT * K // 16, shift, 0, unroll=4)
    idxs = (idx_v, idx2_v)

    def gather(c, buf):
        for b in range(B):
            pltpu.async_copy(
                tab_hbm.at[idxs[b].at[pl.ds(c * ROWS, ROWS)]],
                rows_v.at[buf, pl.ds(b * ROWS, ROWS)],
                gsem.at[buf],
            )

    def gwait(c, buf):
        for b in range(B):
            pltpu.make_async_copy(
                tab_hbm.at[idxs[b].at[pl.ds(c * ROWS, ROWS)]],
                rows_v.at[buf, pl.ds(b * ROWS, ROWS)],
                gsem.at[buf],
            ).wait()

    def compute(c, buf):
        def do_v(v, _):
            coff = (v // 8) * 256 + (v % 8) * 16
            for p in range(CP):
                for b in range(B):
                    acc = None
                    for k in range(K):
                        r = rows_v[buf, b * ROWS + p * K + k, pl.ds(v * 16, 16)]
                        acc = r if acc is None else jnp.maximum(acc, r)
                    stage_v[buf, pl.ds(p * SLAB + b * 128 + coff, 16)] = acc
            return _

        lax.fori_loop(0, 16, do_v, 0, unroll=2)
        pltpu.async_copy(
            stage_v.at[buf],
            out_hbm.at[pl.ds((start_pt + c * CP) * SLAB, CP * SLAB)],
            osem.at[buf],
        )

    def owait(c, buf):
        pltpu.make_async_copy(
            stage_v.at[buf],
            out_hbm.at[pl.ds((start_pt + c * CP) * SLAB, CP * SLAB)],
            osem.at[buf],
        ).wait()

    ncr = jnp.minimum(jnp.maximum((N_OUT - start_pt) // CP, 0), NCH)

    @pl.when(ncr > 0)
    def _prologue():
        gather(0, 0)

    def step(c, _):
        buf = lax.rem(c, 2)
        nbuf = 1 - buf

        @pl.when(c + 1 < ncr)
        def _prefetch():
            gather(c + 1, nbuf)

        gwait(c, buf)

        @pl.when(c >= 2)
        def _drain_out():
            owait(c - 2, buf)

        compute(c, buf)
        return _

    lax.fori_loop(0, ncr, step, 0)

    @pl.when(ncr >= 2)
    def _drain_m2():
        owait(ncr - 2, lax.rem(ncr - 2, 2))

    @pl.when(ncr >= 1)
    def _drain_m1():
        owait(ncr - 1, lax.rem(ncr - 1, 2))


_sc_call = functools.partial(
    pl.kernel,
    out_type=jax.ShapeDtypeStruct((N_OUT * SLAB,), jnp.float32),
    mesh=plsc.VectorSubcoreMesh(core_axis_name="c", subcore_axis_name="s"),
    compiler_params=pltpu.CompilerParams(
        needs_layout_passes=False, use_tc_tiling_on_sc=True
    ),
    scratch_types=[
        pltpu.VMEM((PPT * K,), jnp.int32),
        pltpu.VMEM((PPT * K,), jnp.int32),
        pltpu.VMEM((2, B * ROWS, C), jnp.float32),
        pltpu.VMEM((2, CP * SLAB), jnp.float32),
        pltpu.SemaphoreType.DMA((2,)),
        pltpu.SemaphoreType.DMA((2,)),
    ],
)(_body)


def kernel(x, idx):
    tab = x.transpose(0, 2, 1).reshape(B * N_IN, C)
    idxg = jnp.concatenate(
        [idx.reshape(-1), jnp.zeros((NPAD - N_OUT) * K, jnp.int32)]
    )
    out = _sc_call(tab, idxg)
    o4 = out.reshape(N_OUT, 2, B, 128)
    return o4.transpose(2, 1, 3, 0).reshape(B, C, N_OUT)
